# baseline (speedup 1.0000x reference)
import functools

import jax
import jax.numpy as jnp
from jax import lax
from jax.experimental import pallas as pl
from jax.experimental.pallas import tpu as pltpu
from jax.experimental.pallas import tpu_sc as plsc

B = 16384
D = 32
NC = 2
NS = 16
L = 16
NW = NC * NS
B_PER_W = B // NW
G = 4
N_GROUPS = B_PER_W // G


def _dot_kernel(user_hbm, item_hbm, vu_hbm, vi_hbm, out_hbm,
                idx_uv, idx_iv, win_u, win_i, out_v, sem0, sem1):
    wid = lax.axis_index("s") * NC + lax.axis_index("c")
    base = wid * B_PER_W
    sems = (sem0, sem1)

    pltpu.sync_copy(user_hbm.at[pl.ds(base, B_PER_W)], idx_uv)
    pltpu.sync_copy(item_hbm.at[pl.ds(base, B_PER_W)], idx_iv)

    lanes = lax.iota(jnp.int32, L)
    slot = jnp.minimum(lanes, G - 1)
    lane_mask = lanes < G
    zero = jnp.zeros((L,), jnp.int32)

    def issue(g, phase):
        iu = plsc.load_gather(idx_uv, [g * G + slot])
        ii = plsc.load_gather(idx_iv, [g * G + slot])
        cu = (iu >> 7) * 128
        ci = (ii >> 7) * 128
        for k in range(G):
            c_u = pl.multiple_of(cu[k], 128)
            pltpu.async_copy(
                vu_hbm.at[:, pl.ds(c_u, 128)],
                win_u.at[phase * G + k], sems[phase])
            c_i = pl.multiple_of(ci[k], 128)
            pltpu.async_copy(
                vi_hbm.at[:, pl.ds(c_i, 128)],
                win_i.at[phase * G + k], sems[phase])

    def drain(phase):
        for k in range(G):
            pltpu.make_async_copy(
                vu_hbm.at[:, pl.ds(0, 128)],
                win_u.at[phase * G + k], sems[phase]).wait()
            pltpu.make_async_copy(
                vi_hbm.at[:, pl.ds(0, 128)],
                win_i.at[phase * G + k], sems[phase]).wait()

    def compute(g, phase):
        pos = g * G + slot
        col_u = plsc.load_gather(idx_uv, [pos]) & 127
        col_i = plsc.load_gather(idx_iv, [pos]) & 127
        pslot = phase * G + slot
        acc = (plsc.load_gather(win_u, [pslot, zero, col_u])
               * plsc.load_gather(win_i, [pslot, zero, col_i]))
        for d in range(1, D):
            dv = jnp.full((L,), d, jnp.int32)
            au = plsc.load_gather(win_u, [pslot, dv, col_u])
            ai = plsc.load_gather(win_i, [pslot, dv, col_i])
            acc = acc + au * ai
        plsc.store_scatter(out_v, [pos], acc, mask=lane_mask)

    issue(0, 0)
    issue(1, 1)

    def pair_body(p, _):
        for ph in range(2):
            g = p * 2 + ph
            drain(ph)
            compute(g, ph)

            @pl.when(g + 2 < N_GROUPS)
            def _issue():
                issue(g + 2, ph)
        return _

    lax.fori_loop(0, N_GROUPS // 2, pair_body, 0, unroll=False)

    pltpu.sync_copy(out_v, out_hbm.at[pl.ds(base, B_PER_W)])


@jax.jit
def _run(user, item, vu, vi):
    mesh = plsc.VectorSubcoreMesh(
        core_axis_name="c", subcore_axis_name="s",
        num_cores=NC, num_subcores=NS)
    f = pl.kernel(
        _dot_kernel,
        out_type=jax.ShapeDtypeStruct((B,), jnp.float32),
        mesh=mesh,
        compiler_params=pltpu.CompilerParams(
            needs_layout_passes=False,
            use_tc_tiling_on_sc=True,
        ),
        scratch_types=[
            pltpu.VMEM((B_PER_W,), jnp.int32),
            pltpu.VMEM((B_PER_W,), jnp.int32),
            pltpu.VMEM((2 * G, D, 128), jnp.float32),
            pltpu.VMEM((2 * G, D, 128), jnp.float32),
            pltpu.VMEM((B_PER_W,), jnp.float32),
            pltpu.SemaphoreType.DMA,
            pltpu.SemaphoreType.DMA,
        ],
    )
    return f(user, item, vu, vi)


def kernel(user, item, emb_user, emb_item):
    return _run(user, item, emb_user.T, emb_item.T)


# speedup vs baseline: 1.0102x; 1.0102x over previous
import functools

import jax
import jax.numpy as jnp
from jax import lax
from jax.experimental import pallas as pl
from jax.experimental.pallas import tpu as pltpu
from jax.experimental.pallas import tpu_sc as plsc

B = 16384
D = 32
NC = 2
NS = 16
L = 16
NW = NC * NS
B_PER_W = B // NW
G = 8


def _dot_kernel(user_hbm, item_hbm, vu_hbm, vi_hbm, out_hbm,
                idx_uv, idx_iv, win_u, win_i, out_v, sem):
    wid = lax.axis_index("s") * NC + lax.axis_index("c")
    base = wid * B_PER_W

    pltpu.sync_copy(user_hbm.at[pl.ds(base, B_PER_W)], idx_uv)
    pltpu.sync_copy(item_hbm.at[pl.ds(base, B_PER_W)], idx_iv)

    lanes = lax.iota(jnp.int32, L)
    slot = jnp.minimum(lanes, G - 1)
    lane_mask = lanes < G
    zero = jnp.zeros((L,), jnp.int32)

    def pair_body(p, _):
        iu16 = idx_uv[pl.ds(p * L, L)]
        ii16 = idx_iv[pl.ds(p * L, L)]
        cu16 = (iu16 >> 7) * 128
        ci16 = (ii16 >> 7) * 128

        for half in range(2):
            copies = []
            for k in range(G):
                c_u = pl.multiple_of(cu16[half * G + k], 128)
                copies.append(pltpu.async_copy(
                    vu_hbm.at[:, pl.ds(c_u, 128)], win_u.at[k], sem))
                c_i = pl.multiple_of(ci16[half * G + k], 128)
                copies.append(pltpu.async_copy(
                    vi_hbm.at[:, pl.ds(c_i, 128)], win_i.at[k], sem))
            for cp in copies:
                cp.wait()

            pos = p * L + half * G + slot
            col_u = plsc.load_gather(idx_uv, [pos]) & 127
            col_i = plsc.load_gather(idx_iv, [pos]) & 127
            acc = (plsc.load_gather(win_u, [slot, zero, col_u])
                   * plsc.load_gather(win_i, [slot, zero, col_i]))
            for d in range(1, D):
                dv = jnp.full((L,), d, jnp.int32)
                au = plsc.load_gather(win_u, [slot, dv, col_u])
                ai = plsc.load_gather(win_i, [slot, dv, col_i])
                acc = acc + au * ai
            plsc.store_scatter(out_v, [pos], acc, mask=lane_mask)
        return _

    lax.fori_loop(0, B_PER_W // L, pair_body, 0, unroll=False)

    pltpu.sync_copy(out_v, out_hbm.at[pl.ds(base, B_PER_W)])


@jax.jit
def _run(user, item, vu, vi):
    mesh = plsc.VectorSubcoreMesh(
        core_axis_name="c", subcore_axis_name="s",
        num_cores=NC, num_subcores=NS)
    f = pl.kernel(
        _dot_kernel,
        out_type=jax.ShapeDtypeStruct((B,), jnp.float32),
        mesh=mesh,
        compiler_params=pltpu.CompilerParams(
            needs_layout_passes=False,
            use_tc_tiling_on_sc=True,
        ),
        scratch_types=[
            pltpu.VMEM((B_PER_W,), jnp.int32),
            pltpu.VMEM((B_PER_W,), jnp.int32),
            pltpu.VMEM((G, D, 128), jnp.float32),
            pltpu.VMEM((G, D, 128), jnp.float32),
            pltpu.VMEM((B_PER_W,), jnp.float32),
            pltpu.SemaphoreType.DMA,
        ],
    )
    return f(user, item, vu, vi)


def kernel(user, item, emb_user, emb_item):
    return _run(user, item, emb_user.T, emb_item.T)


# speedup vs baseline: 1.0133x; 1.0031x over previous
import jax
import jax.numpy as jnp
from jax import lax
from jax.experimental import pallas as pl
from jax.experimental.pallas import tpu as pltpu
from jax.experimental.pallas import tpu_sc as plsc

B = 16384
D = 32
NC = 2
NS = 16
L = 16
NW = NC * NS
B_PER_W = B // NW
G = 8


def _dot_kernel(user_hbm, item_hbm, vu_hbm, vi_hbm, out_hbm,
                idx_uv, idx_iv, win_u, win_i, out_v, sem):
    wid = lax.axis_index("s") * NC + lax.axis_index("c")
    base = wid * B_PER_W

    pltpu.sync_copy(user_hbm.at[pl.ds(base, B_PER_W)], idx_uv)
    pltpu.sync_copy(item_hbm.at[pl.ds(base, B_PER_W)], idx_iv)

    lanes = lax.iota(jnp.int32, L)
    slot = jnp.minimum(lanes, G - 1)
    lane_mask = lanes < G
    zero = jnp.zeros((L,), jnp.int32)

    def pair_body(p, _):
        iu16 = idx_uv[pl.ds(p * L, L)]
        ii16 = idx_iv[pl.ds(p * L, L)]
        cu16 = (iu16 >> 7) * 128
        ci16 = (ii16 >> 7) * 128

        for half in range(2):
            copies = []
            for k in range(G):
                c_u = pl.multiple_of(cu16[half * G + k], 128)
                copies.append(pltpu.async_copy(
                    vu_hbm.at[:, pl.ds(c_u, 128)], win_u.at[k], sem))
                c_i = pl.multiple_of(ci16[half * G + k], 128)
                copies.append(pltpu.async_copy(
                    vi_hbm.at[:, pl.ds(c_i, 128)], win_i.at[k], sem))
            for cp in copies:
                cp.wait()

            pos = p * L + half * G + slot
            col_u = plsc.load_gather(idx_uv, [pos]) & 127
            col_i = plsc.load_gather(idx_iv, [pos]) & 127
            acc = (plsc.load_gather(win_u, [slot, zero, col_u])
                   * plsc.load_gather(win_i, [slot, zero, col_i]))
            for d in range(1, D):
                dv = jnp.full((L,), d, jnp.int32)
                au = plsc.load_gather(win_u, [slot, dv, col_u])
                ai = plsc.load_gather(win_i, [slot, dv, col_i])
                acc = acc + au * ai
            plsc.store_scatter(out_v, [pos], acc, mask=lane_mask)
        return _

    lax.fori_loop(0, B_PER_W // L, pair_body, 0, unroll=False)

    pltpu.sync_copy(out_v, out_hbm.at[pl.ds(base, B_PER_W)])


@jax.jit
def _run(user, item, vu, vi):
    mesh = plsc.VectorSubcoreMesh(
        core_axis_name="c", subcore_axis_name="s",
        num_cores=NC, num_subcores=NS)
    f = pl.kernel(
        _dot_kernel,
        out_type=jax.ShapeDtypeStruct((B,), jnp.float32),
        mesh=mesh,
        compiler_params=pltpu.CompilerParams(
            needs_layout_passes=False,
            use_tc_tiling_on_sc=True,
        ),
        scratch_types=[
            pltpu.VMEM((B_PER_W,), jnp.int32),
            pltpu.VMEM((B_PER_W,), jnp.int32),
            pltpu.VMEM((G, D, 128), jnp.float32),
            pltpu.VMEM((G, D, 128), jnp.float32),
            pltpu.VMEM((B_PER_W,), jnp.float32),
            pltpu.SemaphoreType.DMA,
        ],
    )
    return f(user, item, vu, vi)


def kernel(user, item, emb_user, emb_item):
    return _run(user, item, emb_user.T, emb_item.T)
